# Initial kernel scaffold; baseline (speedup 1.0000x reference)
import jax
import jax.numpy as jnp
from jax.experimental import pallas as pl


def kernel(x1, x2, w1, b1, w2, b2, g1, be1, g2, be2):
    raise NotImplementedError("write your pallas kernel here")


# speedup vs baseline: 1.0672x; 1.0672x over previous
import functools

import jax
import jax.numpy as jnp
from jax.experimental import pallas as pl
from jax.experimental.pallas import tpu as pltpu


def _stats1_kernel(x_ref, w1_ref, b1_ref, st_ref):
    h = jnp.dot(w1_ref[...], x_ref[0], preferred_element_type=jnp.float32)
    h = h + b1_ref[...]
    st_ref[0] = jnp.concatenate(
        [jnp.sum(h, axis=-1, keepdims=True),
         jnp.sum(h * h, axis=-1, keepdims=True)], axis=-1)


def _body_kernel(x_ref, w1_ref, b1_ref, sc1_ref, sh1_ref, w2_ref, b2_ref,
                 y_ref, st_ref, *, ksize, stride, pad, l_in, l_out, m_max):
    c_out = w1_ref.shape[0]
    w = max(l_in, m_max)

    h = jnp.dot(w1_ref[...], x_ref[0], preferred_element_type=jnp.float32)
    hn = jnp.maximum((h + b1_ref[...]) * sc1_ref[...] + sh1_ref[...], 0.0)
    if w > l_in:
        hn = jnp.concatenate(
            [hn, jnp.zeros((c_out, w - l_in), jnp.float32)], axis=-1)
    zall = jnp.dot(w2_ref[...], hn.astype(jnp.bfloat16),
                   preferred_element_type=jnp.float32)

    lane = jax.lax.broadcasted_iota(jnp.int32, (c_out, w), 1)
    b2 = b2_ref[...]
    s1 = jnp.zeros((c_out, 1), jnp.float32)
    s2 = jnp.zeros((c_out, 1), jnp.float32)
    for p in range(stride):
        m_p = (l_out - p + stride - 1) // stride
        acc = jnp.zeros((c_out, w), jnp.float32)
        for k in range(ksize):
            d = k - pad
            if d % stride != p:
                continue
            q = (d - p) // stride
            m_lo, m_hi = max(0, q), min(m_p, l_in + q)
            if m_hi <= m_lo:
                continue
            zk = zall[k * c_out:(k + 1) * c_out]
            if q % w:
                zk = pltpu.roll(zk, shift=q % w, axis=1)
            if m_lo == 0 and m_hi == w:
                acc = acc + zk
            else:
                acc = acc + jnp.where((lane >= m_lo) & (lane < m_hi), zk, 0.0)
        y_p = jnp.where(lane < m_p, acc + b2, 0.0)
        s1 = s1 + jnp.sum(y_p, axis=-1, keepdims=True)
        s2 = s2 + jnp.sum(y_p * y_p, axis=-1, keepdims=True)
        y_ref[0, p] = (y_p[:, :m_max] if w > m_max else y_p).astype(jnp.bfloat16)

    st_ref[0] = jnp.concatenate([s1, s2], axis=-1)


def _l_tile(l, cap=2048):
    if l <= cap:
        return l
    for t in range(cap - cap % 128, 127, -128):
        if l % t == 0:
            return t
    return l


def kernel(x1, x2, w1, b1, w2, b2, g1, be1, g2, be2):
    eps = 1e-5
    stride = 2
    ksize = w2.shape[2]
    n, c1, l1 = x1.shape
    _, c2, l2 = x2.shape
    c_in = c1 + c2
    c_out = w1.shape[0]
    pad = stride // 2
    l_out = (l2 - 1) * stride - 2 * pad + ksize
    m_max = -(-l_out // stride)

    diff = l2 - l1
    x1p = jnp.pad(x1, ((0, 0), (0, 0), (diff // 2, diff - diff // 2)))
    x = jnp.concatenate([x1p, x2], axis=1).astype(jnp.bfloat16)

    w1m = w1[:, :, 0].astype(jnp.bfloat16)
    b1c = b1.reshape(c_out, 1)
    w2all = jnp.transpose(w2, (2, 1, 0)).reshape(ksize * c_out, c_out)
    w2all = w2all.astype(jnp.bfloat16)
    b2c = b2.reshape(c_out, 1)

    lt = _l_tile(l2)
    n_t = l2 // lt
    stats1 = pl.pallas_call(
        _stats1_kernel,
        out_shape=jax.ShapeDtypeStruct((n * n_t, c_out, 2), jnp.float32),
        grid=(n, n_t),
        in_specs=[
            pl.BlockSpec((1, c_in, lt), lambda i, t: (i, 0, t)),
            pl.BlockSpec((c_out, c_in), lambda i, t: (0, 0)),
            pl.BlockSpec((c_out, 1), lambda i, t: (0, 0)),
        ],
        out_specs=pl.BlockSpec((1, c_out, 2), lambda i, t: (i * n_t + t, 0, 0)),
        compiler_params=pltpu.CompilerParams(
            dimension_semantics=("parallel", "parallel")),
    )(x, w1m, b1c)

    cnt1 = float(n * l2)
    mean1 = jnp.sum(stats1[:, :, 0], axis=0) / cnt1
    var1 = jnp.maximum(jnp.sum(stats1[:, :, 1], axis=0) / cnt1 - mean1 * mean1, 0.0)
    inv1 = jax.lax.rsqrt(var1 + eps)
    scale1 = (g1 * inv1).reshape(c_out, 1)
    shift1 = (be1 - mean1 * g1 * inv1).reshape(c_out, 1)

    body = functools.partial(
        _body_kernel, ksize=ksize, stride=stride, pad=pad,
        l_in=l2, l_out=l_out, m_max=m_max)
    y_phase, stats2 = pl.pallas_call(
        body,
        out_shape=(jax.ShapeDtypeStruct((n, stride, c_out, m_max), jnp.bfloat16),
                   jax.ShapeDtypeStruct((n, c_out, 2), jnp.float32)),
        grid=(n,),
        in_specs=[
            pl.BlockSpec((1, c_in, l2), lambda i: (i, 0, 0)),
            pl.BlockSpec((c_out, c_in), lambda i: (0, 0)),
            pl.BlockSpec((c_out, 1), lambda i: (0, 0)),
            pl.BlockSpec((c_out, 1), lambda i: (0, 0)),
            pl.BlockSpec((c_out, 1), lambda i: (0, 0)),
            pl.BlockSpec((ksize * c_out, c_out), lambda i: (0, 0)),
            pl.BlockSpec((c_out, 1), lambda i: (0, 0)),
        ],
        out_specs=(pl.BlockSpec((1, stride, c_out, m_max), lambda i: (i, 0, 0, 0)),
                   pl.BlockSpec((1, c_out, 2), lambda i: (i, 0, 0))),
        compiler_params=pltpu.CompilerParams(dimension_semantics=("parallel",)),
    )(x, w1m, b1c, scale1, shift1, w2all, b2c)

    cnt2 = float(n * l_out)
    mean2 = jnp.sum(stats2[:, :, 0], axis=0) / cnt2
    var2 = jnp.maximum(jnp.sum(stats2[:, :, 1], axis=0) / cnt2 - mean2 * mean2, 0.0)
    inv2 = jax.lax.rsqrt(var2 + eps)
    scale2 = (g2 * inv2).reshape(1, 1, c_out, 1)
    shift2 = (be2 - mean2 * g2 * inv2).reshape(1, 1, c_out, 1)

    y = jnp.maximum(y_phase.astype(jnp.float32) * scale2 + shift2, 0.0)
    out = jnp.transpose(y, (0, 2, 3, 1)).reshape(n, c_out, m_max * stride)
    return out[:, :, :l_out]
